# baseline (speedup 1.0000x reference)
import functools

import jax
import jax.numpy as jnp
from jax import lax
from jax.experimental import pallas as pl
from jax.experimental.pallas import tpu as pltpu
from jax.experimental.pallas import tpu_sc as plsc

N = 100000
T = 64
B = 64
D = 128

NC = 2
NS = 16
NW = NC * NS
CH = 160
NCHUNK = N // CH
RG = CH // 16
MAXG = 10
HU = 4
DU = 4


def _sqrt16(x):
    i = plsc.bitcast(x, jnp.int32)
    y = plsc.bitcast(jnp.int32(0x5F3759DF) - (i >> 1), jnp.float32)
    for _ in range(3):
        y = y * (1.5 - 0.5 * x * y * y)
    return jnp.where(x > 0.0, x * y, 0.0)


def _sc_body(to_hbm, po_hbm, th_hbm, ph_hbm, hid_hbm, ba_hbm, len_hbm,
             out_hbm, len_v,
             to_b0, to_b1, po_b0, po_b1, th_b0, th_b1, ph_b0, ph_b1,
             hid_b0, hid_b1, ba_b0, ba_b1, acc, sem0, sem1):
    wid = lax.axis_index("s") * NC + lax.axis_index("c")
    niter = (NCHUNK - wid + NW - 1) // NW

    to_b = (to_b0, to_b1)
    po_b = (po_b0, po_b1)
    th_b = (th_b0, th_b1)
    ph_b = (ph_b0, ph_b1)
    hid_b = (hid_b0, hid_b1)
    ba_b = (ba_b0, ba_b1)
    sems = (sem0, sem1)

    zeros16 = jnp.zeros((16,), jnp.float32)
    acc[0, :] = zeros16
    acc[1, :] = zeros16
    acc[2, :] = zeros16
    pltpu.sync_copy(len_hbm, len_v)

    def issue(b, c):
        base = c * CH
        sem = sems[b]
        pltpu.async_copy(to_hbm.at[pl.ds(base, CH)], to_b[b], sem)
        pltpu.async_copy(po_hbm.at[pl.ds(base, CH)], po_b[b], sem)
        pltpu.async_copy(ba_hbm.at[pl.ds(base, CH)], ba_b[b], sem)
        pltpu.async_copy(th_hbm.at[pl.ds(base * T, CH * T)], th_b[b], sem)
        pltpu.async_copy(ph_hbm.at[pl.ds(base * T, CH * T)], ph_b[b], sem)
        pltpu.async_copy(hid_hbm.at[pl.ds(base * D, CH * D)], hid_b[b], sem)

    def drain(b):
        sem = sems[b]
        pltpu.make_async_copy(to_hbm.at[pl.ds(0, CH)], to_b[b], sem).wait()
        pltpu.make_async_copy(po_hbm.at[pl.ds(0, CH)], po_b[b], sem).wait()
        pltpu.make_async_copy(ba_hbm.at[pl.ds(0, CH)], ba_b[b], sem).wait()
        pltpu.make_async_copy(th_hbm.at[pl.ds(0, CH * T)], th_b[b], sem).wait()
        pltpu.make_async_copy(ph_hbm.at[pl.ds(0, CH * T)], ph_b[b], sem).wait()
        pltpu.make_async_copy(hid_hbm.at[pl.ds(0, CH * D)], hid_b[b], sem).wait()

    rows_iota = lax.broadcasted_iota(jnp.int32, (16,), 0)

    def process(b):
        thb, phb, hidb = th_b[b], ph_b[b], hid_b[b]
        tob, pob, bab = to_b[b], po_b[b], ba_b[b]

        def rowgroup(rg, carry):
            ao, ah, an = carry
            r0 = rg * 16

            tov = tob[pl.ds(r0, 16)]
            pov = pob[pl.ds(r0, 16)]
            d0 = pov - tov
            ao = ao + d0 * d0

            bav = bab[pl.ds(r0, 16)]
            lenv = plsc.load_gather(len_v, [bav])
            rbase_t = (r0 + rows_iota) * T

            def hcol(tc, ahc):
                for k in range(HU):
                    t = tc * HU + k
                    idx = rbase_t + t
                    tv = plsc.load_gather(thb, [idx])
                    pv = plsc.load_gather(phb, [idx])
                    dd = pv - tv
                    ahc = ahc + jnp.where(lenv > t, dd * dd, 0.0)
                return ahc

            ah = lax.fori_loop(0, T // HU, hcol, ah)

            rbase_d = (r0 + rows_iota) * D

            def dcol(tc, anc):
                for k in range(DU):
                    idx = rbase_d + tc * DU + k
                    hv = plsc.load_gather(hidb, [idx])
                    anc = anc + hv * hv
                return anc

            ss = lax.fori_loop(0, D // DU, dcol, zeros16)
            an = an + _sqrt16(ss)
            return ao, ah, an

        ao, ah, an = lax.fori_loop(0, RG, rowgroup,
                                   (zeros16, zeros16, zeros16))
        plsc.addupdate(acc.at[0], ao)
        plsc.addupdate(acc.at[1], ah)
        plsc.addupdate(acc.at[2], an)

    issue(0, wid)
    issue(1, wid + NW)

    def gstep(g, carry):
        for bb in (0, 1):
            i = 2 * g + bb
            c = wid + i * NW

            @pl.when(i < niter)
            def _():
                drain(bb)
                process(bb)

                @pl.when(i + 2 < niter)
                def _():
                    issue(bb, c + 2 * NW)

        return carry

    lax.fori_loop(0, MAXG, gstep, 0)
    pltpu.sync_copy(acc, out_hbm.at[wid])


def _sc_losses(truth_out, pred_out, truth_hint, pred_hint, hidden,
               batch_assign, length):
    mesh = plsc.VectorSubcoreMesh(core_axis_name="c", subcore_axis_name="s")
    run = functools.partial(
        pl.kernel,
        out_type=jax.ShapeDtypeStruct((NW, 3, 16), jnp.float32),
        mesh=mesh,
        compiler_params=pltpu.CompilerParams(needs_layout_passes=False),
        scratch_types=[
            pltpu.VMEM((B,), jnp.int32),
            pltpu.VMEM((CH,), jnp.float32),
            pltpu.VMEM((CH,), jnp.float32),
            pltpu.VMEM((CH,), jnp.float32),
            pltpu.VMEM((CH,), jnp.float32),
            pltpu.VMEM((CH * T,), jnp.float32),
            pltpu.VMEM((CH * T,), jnp.float32),
            pltpu.VMEM((CH * T,), jnp.float32),
            pltpu.VMEM((CH * T,), jnp.float32),
            pltpu.VMEM((CH * D,), jnp.float32),
            pltpu.VMEM((CH * D,), jnp.float32),
            pltpu.VMEM((CH,), jnp.int32),
            pltpu.VMEM((CH,), jnp.int32),
            pltpu.VMEM((3, 16), jnp.float32),
            pltpu.SemaphoreType.DMA,
            pltpu.SemaphoreType.DMA,
        ],
    )(_sc_body)
    return run(truth_out, pred_out, truth_hint.reshape(-1),
               pred_hint.reshape(-1), hidden.reshape(-1), batch_assign,
               length)


def kernel(truth_out, pred_out, truth_hint, pred_hint, hidden,
           edge_index, batch_assign, length):
    del edge_index
    parts = _sc_losses(truth_out, pred_out, truth_hint, pred_hint, hidden,
                       batch_assign, length)
    sums = jnp.sum(parts, axis=(0, 2))
    output_loss = (sums[0] / N).reshape(1)
    hint_loss = (sums[1] / (N * T)).reshape(1)
    hidden_loss = sums[2] / N
    return (output_loss, hint_loss, hidden_loss)


# speedup vs baseline: 2.5079x; 2.5079x over previous
import functools

import jax
import jax.numpy as jnp
from jax import lax
from jax.experimental import pallas as pl
from jax.experimental.pallas import tpu as pltpu
from jax.experimental.pallas import tpu_sc as plsc

N = 100000
T = 64
B = 64
D = 128

NC = 2
NS = 16
NW = NC * NS
CH = 160
NCHUNK = N // CH
RG = CH // 16
MAXG = 10
HU = 4
DU = 4


def _sqrt16(x):
    i = plsc.bitcast(x, jnp.int32)
    y = plsc.bitcast(jnp.int32(0x5F3759DF) - (i >> 1), jnp.float32)
    for _ in range(3):
        y = y * (1.5 - 0.5 * x * y * y)
    return jnp.where(x > 0.0, x * y, 0.0)


def _sc_body(to_hbm, po_hbm, th_hbm, ph_hbm, hid_hbm, ba_hbm, len_hbm,
             out_hbm, len_v,
             to_b0, to_b1, po_b0, po_b1, th_b0, th_b1, ph_b0, ph_b1,
             hid_b0, hid_b1, ba_b0, ba_b1, acc, sem0, sem1):
    wid = lax.axis_index("s") * NC + lax.axis_index("c")
    niter = (NCHUNK - wid + NW - 1) // NW

    to_b = (to_b0, to_b1)
    po_b = (po_b0, po_b1)
    th_b = (th_b0, th_b1)
    ph_b = (ph_b0, ph_b1)
    hid_b = (hid_b0, hid_b1)
    ba_b = (ba_b0, ba_b1)
    sems = (sem0, sem1)

    zeros16 = jnp.zeros((16,), jnp.float32)
    acc[0, :] = zeros16
    acc[1, :] = zeros16
    acc[2, :] = zeros16
    pltpu.sync_copy(len_hbm, len_v)

    def issue(b, c):
        base = c * CH
        sem = sems[b]
        pltpu.async_copy(to_hbm.at[pl.ds(base, CH)], to_b[b], sem)
        pltpu.async_copy(po_hbm.at[pl.ds(base, CH)], po_b[b], sem)
        pltpu.async_copy(ba_hbm.at[pl.ds(base, CH)], ba_b[b], sem)
        pltpu.async_copy(th_hbm.at[pl.ds(base * T, CH * T)], th_b[b], sem)
        pltpu.async_copy(ph_hbm.at[pl.ds(base * T, CH * T)], ph_b[b], sem)
        pltpu.async_copy(hid_hbm.at[pl.ds(base * D, CH * D)], hid_b[b], sem)

    def drain(b):
        sem = sems[b]
        pltpu.make_async_copy(to_hbm.at[pl.ds(0, CH)], to_b[b], sem).wait()
        pltpu.make_async_copy(po_hbm.at[pl.ds(0, CH)], po_b[b], sem).wait()
        pltpu.make_async_copy(ba_hbm.at[pl.ds(0, CH)], ba_b[b], sem).wait()
        pltpu.make_async_copy(th_hbm.at[pl.ds(0, CH * T)], th_b[b], sem).wait()
        pltpu.make_async_copy(ph_hbm.at[pl.ds(0, CH * T)], ph_b[b], sem).wait()
        pltpu.make_async_copy(hid_hbm.at[pl.ds(0, CH * D)], hid_b[b], sem).wait()

    rows_iota = lax.broadcasted_iota(jnp.int32, (16,), 0)

    def process(b):
        thb, phb, hidb = th_b[b], ph_b[b], hid_b[b]
        tob, pob, bab = to_b[b], po_b[b], ba_b[b]

        def rowgroup(rg, carry):
            ao, ah, an = carry
            r0 = rg * 16

            tov = tob[pl.ds(r0, 16)]
            pov = pob[pl.ds(r0, 16)]
            d0 = pov - tov
            ao = ao + d0 * d0

            bav = bab[pl.ds(r0, 16)]
            lenv = plsc.load_gather(len_v, [bav])
            rbase_t = (r0 + rows_iota) * T

            def hcol(tc, ahc):
                for k in range(HU):
                    t = tc * HU + k
                    colv = (rows_iota + t) & (T - 1)
                    idx = rbase_t + colv
                    tv = plsc.load_gather(thb, [idx])
                    pv = plsc.load_gather(phb, [idx])
                    dd = pv - tv
                    ahc = ahc + jnp.where(lenv > colv, dd * dd, 0.0)
                return ahc

            ah = lax.fori_loop(0, T // HU, hcol, ah)

            rbase_d = (r0 + rows_iota) * D

            def dcol(tc, anc):
                for k in range(DU):
                    c = tc * DU + k
                    idx = rbase_d + ((rows_iota + c) & (D - 1))
                    hv = plsc.load_gather(hidb, [idx])
                    anc = anc + hv * hv
                return anc

            ss = lax.fori_loop(0, D // DU, dcol, zeros16)
            an = an + _sqrt16(ss)
            return ao, ah, an

        ao, ah, an = lax.fori_loop(0, RG, rowgroup,
                                   (zeros16, zeros16, zeros16))
        plsc.addupdate(acc.at[0], ao)
        plsc.addupdate(acc.at[1], ah)
        plsc.addupdate(acc.at[2], an)

    issue(0, wid)
    issue(1, wid + NW)

    def gstep(g, carry):
        for bb in (0, 1):
            i = 2 * g + bb
            c = wid + i * NW

            @pl.when(i < niter)
            def _():
                drain(bb)
                process(bb)

                @pl.when(i + 2 < niter)
                def _():
                    issue(bb, c + 2 * NW)

        return carry

    lax.fori_loop(0, MAXG, gstep, 0)
    pltpu.sync_copy(acc, out_hbm.at[wid])


def _sc_losses(truth_out, pred_out, truth_hint, pred_hint, hidden,
               batch_assign, length):
    mesh = plsc.VectorSubcoreMesh(core_axis_name="c", subcore_axis_name="s")
    run = functools.partial(
        pl.kernel,
        out_type=jax.ShapeDtypeStruct((NW, 3, 16), jnp.float32),
        mesh=mesh,
        compiler_params=pltpu.CompilerParams(needs_layout_passes=False),
        scratch_types=[
            pltpu.VMEM((B,), jnp.int32),
            pltpu.VMEM((CH,), jnp.float32),
            pltpu.VMEM((CH,), jnp.float32),
            pltpu.VMEM((CH,), jnp.float32),
            pltpu.VMEM((CH,), jnp.float32),
            pltpu.VMEM((CH * T,), jnp.float32),
            pltpu.VMEM((CH * T,), jnp.float32),
            pltpu.VMEM((CH * T,), jnp.float32),
            pltpu.VMEM((CH * T,), jnp.float32),
            pltpu.VMEM((CH * D,), jnp.float32),
            pltpu.VMEM((CH * D,), jnp.float32),
            pltpu.VMEM((CH,), jnp.int32),
            pltpu.VMEM((CH,), jnp.int32),
            pltpu.VMEM((3, 16), jnp.float32),
            pltpu.SemaphoreType.DMA,
            pltpu.SemaphoreType.DMA,
        ],
    )(_sc_body)
    return run(truth_out, pred_out, truth_hint.reshape(-1),
               pred_hint.reshape(-1), hidden.reshape(-1), batch_assign,
               length)


def kernel(truth_out, pred_out, truth_hint, pred_hint, hidden,
           edge_index, batch_assign, length):
    del edge_index
    parts = _sc_losses(truth_out, pred_out, truth_hint, pred_hint, hidden,
                       batch_assign, length)
    sums = jnp.sum(parts, axis=(0, 2))
    output_loss = (sums[0] / N).reshape(1)
    hint_loss = (sums[1] / (N * T)).reshape(1)
    hidden_loss = sums[2] / N
    return (output_loss, hint_loss, hidden_loss)


# speedup vs baseline: 3.1308x; 1.2484x over previous
import functools

import jax
import jax.numpy as jnp
from jax import lax
from jax.experimental import pallas as pl
from jax.experimental.pallas import tpu as pltpu
from jax.experimental.pallas import tpu_sc as plsc

N = 100000
T = 64
B = 64
D = 128

NC = 2
NS = 16
NW = NC * NS
CH = 160
NCHUNK = N // CH
RG = CH // 16
MAXG = 10
HU = 4
DU = 4


def _sqrt16(x):
    i = plsc.bitcast(x, jnp.int32)
    y = plsc.bitcast(jnp.int32(0x5F3759DF) - (i >> 1), jnp.float32)
    for _ in range(3):
        y = y * (1.5 - 0.5 * x * y * y)
    return jnp.where(x > 0.0, x * y, 0.0)


def _sc_body(to_hbm, po_hbm, th_hbm, ph_hbm, hid_hbm, ba_hbm, len_hbm,
             out_hbm, len_v,
             to_b0, to_b1, po_b0, po_b1, th_b0, th_b1, ph_b0, ph_b1,
             hid_b0, hid_b1, ba_b0, ba_b1, acc, sem0, sem1):
    wid = lax.axis_index("s") * NC + lax.axis_index("c")
    niter = (NCHUNK - wid + NW - 1) // NW

    to_b = (to_b0, to_b1)
    po_b = (po_b0, po_b1)
    th_b = (th_b0, th_b1)
    ph_b = (ph_b0, ph_b1)
    hid_b = (hid_b0, hid_b1)
    ba_b = (ba_b0, ba_b1)
    sems = (sem0, sem1)

    zeros16 = jnp.zeros((16,), jnp.float32)
    acc[0, :] = zeros16
    acc[1, :] = zeros16
    acc[2, :] = zeros16
    pltpu.sync_copy(len_hbm, len_v)

    def issue(b, c):
        base = c * CH
        sem = sems[b]
        pltpu.async_copy(to_hbm.at[pl.ds(base, CH)], to_b[b], sem)
        pltpu.async_copy(po_hbm.at[pl.ds(base, CH)], po_b[b], sem)
        pltpu.async_copy(ba_hbm.at[pl.ds(base, CH)], ba_b[b], sem)
        pltpu.async_copy(th_hbm.at[pl.ds(base, CH)], th_b[b], sem)
        pltpu.async_copy(ph_hbm.at[pl.ds(base, CH)], ph_b[b], sem)
        pltpu.async_copy(hid_hbm.at[pl.ds(base, CH)], hid_b[b], sem)

    def drain(b):
        sem = sems[b]
        pltpu.make_async_copy(to_hbm.at[pl.ds(0, CH)], to_b[b], sem).wait()
        pltpu.make_async_copy(po_hbm.at[pl.ds(0, CH)], po_b[b], sem).wait()
        pltpu.make_async_copy(ba_hbm.at[pl.ds(0, CH)], ba_b[b], sem).wait()
        pltpu.make_async_copy(th_hbm.at[pl.ds(0, CH)], th_b[b], sem).wait()
        pltpu.make_async_copy(ph_hbm.at[pl.ds(0, CH)], ph_b[b], sem).wait()
        pltpu.make_async_copy(hid_hbm.at[pl.ds(0, CH)], hid_b[b], sem).wait()

    rows_iota = lax.broadcasted_iota(jnp.int32, (16,), 0)

    def process(b):
        thb, phb, hidb = th_b[b], ph_b[b], hid_b[b]
        tob, pob, bab = to_b[b], po_b[b], ba_b[b]

        def rowgroup(rg, carry):
            ao, ah, an = carry
            r0 = rg * 16

            tov = tob[pl.ds(r0, 16)]
            pov = pob[pl.ds(r0, 16)]
            d0 = pov - tov
            ao = ao + d0 * d0

            bav = bab[pl.ds(r0, 16)]
            lenv = plsc.load_gather(len_v, [bav])
            rowv = r0 + rows_iota

            def hcol(tc, ahc):
                for k in range(HU):
                    t = tc * HU + k
                    colv = (rows_iota + t) & (T - 1)
                    tv = plsc.load_gather(thb, [rowv, colv])
                    pv = plsc.load_gather(phb, [rowv, colv])
                    dd = pv - tv
                    ahc = ahc + jnp.where(lenv > colv, dd * dd, 0.0)
                return ahc

            ah = lax.fori_loop(0, T // HU, hcol, ah)

            def dcol(tc, anc):
                for k in range(DU):
                    c = tc * DU + k
                    colv = (rows_iota + c) & (D - 1)
                    hv = plsc.load_gather(hidb, [rowv, colv])
                    anc = anc + hv * hv
                return anc

            ss = lax.fori_loop(0, D // DU, dcol, zeros16)
            an = an + _sqrt16(ss)
            return ao, ah, an

        ao, ah, an = lax.fori_loop(0, RG, rowgroup,
                                   (zeros16, zeros16, zeros16))
        plsc.addupdate(acc.at[0], ao)
        plsc.addupdate(acc.at[1], ah)
        plsc.addupdate(acc.at[2], an)

    issue(0, wid)
    issue(1, wid + NW)

    def gstep(g, carry):
        for bb in (0, 1):
            i = 2 * g + bb
            c = wid + i * NW

            @pl.when(i < niter)
            def _():
                drain(bb)
                process(bb)

                @pl.when(i + 2 < niter)
                def _():
                    issue(bb, c + 2 * NW)

        return carry

    lax.fori_loop(0, MAXG, gstep, 0)
    pltpu.sync_copy(acc, out_hbm.at[wid])


def _sc_losses(truth_out, pred_out, truth_hint, pred_hint, hidden,
               batch_assign, length):
    mesh = plsc.VectorSubcoreMesh(core_axis_name="c", subcore_axis_name="s")
    run = functools.partial(
        pl.kernel,
        out_type=jax.ShapeDtypeStruct((NW, 3, 16), jnp.float32),
        mesh=mesh,
        compiler_params=pltpu.CompilerParams(needs_layout_passes=False),
        scratch_types=[
            pltpu.VMEM((B,), jnp.int32),
            pltpu.VMEM((CH,), jnp.float32),
            pltpu.VMEM((CH,), jnp.float32),
            pltpu.VMEM((CH,), jnp.float32),
            pltpu.VMEM((CH,), jnp.float32),
            pltpu.VMEM((CH, T), jnp.float32),
            pltpu.VMEM((CH, T), jnp.float32),
            pltpu.VMEM((CH, T), jnp.float32),
            pltpu.VMEM((CH, T), jnp.float32),
            pltpu.VMEM((CH, D), jnp.float32),
            pltpu.VMEM((CH, D), jnp.float32),
            pltpu.VMEM((CH,), jnp.int32),
            pltpu.VMEM((CH,), jnp.int32),
            pltpu.VMEM((3, 16), jnp.float32),
            pltpu.SemaphoreType.DMA,
            pltpu.SemaphoreType.DMA,
        ],
    )(_sc_body)
    return run(truth_out, pred_out, truth_hint, pred_hint, hidden,
               batch_assign, length)


def kernel(truth_out, pred_out, truth_hint, pred_hint, hidden,
           edge_index, batch_assign, length):
    del edge_index
    parts = _sc_losses(truth_out, pred_out, truth_hint, pred_hint, hidden,
                       batch_assign, length)
    sums = jnp.sum(parts, axis=(0, 2))
    output_loss = (sums[0] / N).reshape(1)
    hint_loss = (sums[1] / (N * T)).reshape(1)
    hidden_loss = sums[2] / N
    return (output_loss, hint_loss, hidden_loss)
